# baseline (speedup 1.0000x reference)
import functools

import jax
import jax.numpy as jnp
from jax import lax
from jax.experimental import pallas as pl
from jax.experimental.pallas import tpu as pltpu
from jax.experimental.pallas import tpu_sc as plsc

_SEG = 1024
_ROW = 240
_SLOT = 16 * _ROW


def _score_body(x_ref, w1_ref, b1_ref, w2_ref, b2_ref, out_ref):
    x = x_ref[...]
    h = jnp.dot(x, w1_ref[...], preferred_element_type=jnp.float32) + b1_ref[...]
    h = jnp.maximum(h, 0.0)
    out_ref[...] = jnp.dot(h, w2_ref[...], preferred_element_type=jnp.float32) + b2_ref[...]


def _apply_body(x_ref, s_ref, o_ref):
    o_ref[...] = x_ref[...] * s_ref[...]


def _sc_select_body(l_hbm, sel_hbm, ek_hbm,
                    l_v, e_v, sb_v, tp_v, sel_v, stbig_v, redbig_v, ek_v, part):
    c = lax.axis_index("c")
    s = lax.axis_index("s")
    rg = s // 8
    seg = s % 8
    row = c * 2 + rg
    base = rg * 8
    off = seg * _SEG
    lane = lax.iota(jnp.int32, 16)
    zero16 = jnp.zeros((16,), jnp.float32)

    def _allmax(v):
        for sh in (1, 2, 4, 8):
            v = jnp.maximum(v, jnp.take(v, lane ^ sh))
        return v

    def _allsum(v):
        for sh in (1, 2, 4, 8):
            v = v + jnp.take(v, lane ^ sh)
        return v

    def _staged(accs, slot, op):
        n = len(accs)
        for j, a in enumerate(accs):
            stbig_v[pl.ds(j * 16, 16)] = a
        pltpu.sync_copy(stbig_v.at[pl.ds(0, n * 16)],
                        part.at[pl.ds(slot * _SLOT + s * _ROW, n * 16)])
        plsc.subcore_barrier()
        pltpu.sync_copy(part.at[pl.ds(slot * _SLOT + base * _ROW, 8 * _ROW)],
                        redbig_v)
        outs = []
        for j in range(n):
            a = redbig_v[pl.ds(j * 16, 16)]
            for g in range(1, 8):
                a = op(a, redbig_v[pl.ds(g * _ROW + j * 16, 16)])
            outs.append(_allsum(a) if op is jnp.add else _allmax(a))
        return outs

    pltpu.sync_copy(l_hbm.at[row, pl.ds(off, _SEG)], l_v)

    def mx_body(i, acc):
        return jnp.maximum(acc, l_v[pl.ds(i * 16, 16)])

    mx = lax.fori_loop(0, _SEG // 16, mx_body,
                       jnp.full((16,), -jnp.inf, jnp.float32))
    (m16,) = _staged([mx], 0, jnp.maximum)

    def ez_body(i, carry):
        zacc, gacc = carry
        x = l_v[pl.ds(i * 16, 16)]
        e = jnp.exp(x - m16)
        e_v[pl.ds(i * 16, 16)] = e
        en = jnp.exp(-jnp.abs(x))
        sig = jnp.where(x >= 0, 1.0 / (1.0 + en), en / (1.0 + en))
        return zacc + e, gacc + sig

    zacc, gacc = lax.fori_loop(0, _SEG // 16, ez_body, (zero16, zero16))
    z16, sigsum16 = _staged([zacc, gacc], 1, jnp.add)
    k16 = sigsum16.astype(jnp.int32).astype(jnp.float32)

    def sbb(i, carry):
        e = e_v[pl.ds(i * 16, 16)]
        sb_v[pl.ds(i * 16, 16)] = lax.bitcast_convert_type(e / z16, jnp.int32)
        return carry

    lax.fori_loop(0, _SEG // 16, sbb, 0)

    P = jnp.zeros((16,), jnp.int32)
    slot = 0
    for sh, nmax in [(28, 7)] + [(24 - 4 * i, 15) for i in range(7)]:
        probes = [P + (n << sh) for n in range(1, nmax + 1)]

        def cbody(i, accs, probes=probes):
            for u in range(4):
                v = sb_v[pl.ds(i * 64 + u * 16, 16)]
                accs = tuple(a + jnp.where(v >= t, 1.0, 0.0)
                             for a, t in zip(accs, probes))
            return accs

        accs = lax.fori_loop(0, _SEG // 64, cbody, (zero16,) * nmax)
        cnts = _staged(list(accs), slot, jnp.add)
        slot = 1 - slot
        nsel = jnp.zeros((16,), jnp.int32)
        for cn in cnts:
            nsel = nsel + jnp.where(cn >= k16, 1, 0)
        P = P + (nsel << sh)
    thr = P

    def gt_body(i, acc):
        v = sb_v[pl.ds(i * 16, 16)]
        p = off + i * 16 + lane
        tp_v[pl.ds(i * 16, 16)] = jnp.where(v == thr, p, 8192)
        return acc + jnp.where(v > thr, 1.0, 0.0)

    (n_gt,) = _staged([lax.fori_loop(0, _SEG // 16, gt_body, zero16)],
                      slot, jnp.add)
    slot = 1 - slot

    Q = jnp.zeros((16,), jnp.int32)
    for sh, nmax in [(12, 1), (8, 15), (4, 15), (0, 15)]:
        probes = [Q + (n << sh) for n in range(1, nmax + 1)]

        def qbody(i, accs, probes=probes):
            for u in range(4):
                tp = tp_v[pl.ds(i * 64 + u * 16, 16)]
                accs = tuple(a + jnp.where(tp < t, 1.0, 0.0)
                             for a, t in zip(accs, probes))
            return accs

        accs = lax.fori_loop(0, _SEG // 64, qbody, (zero16,) * nmax)
        cnts = _staged(list(accs), slot, jnp.add)
        slot = 1 - slot
        nsel = jnp.zeros((16,), jnp.int32)
        for cn in cnts:
            nsel = nsel + jnp.where(n_gt + cn < k16, 1, 0)
        Q = Q + (nsel << sh)
    limit = Q + 1

    def wr_body(i, carry):
        sb = sb_v[pl.ds(i * 16, 16)]
        sf = lax.bitcast_convert_type(sb, jnp.float32)
        p = off + i * 16 + lane
        hardb = (sb > thr) | ((sb == thr) & (p < limit))
        hard = jnp.where(hardb, 1.0, 0.0)
        sel_v[pl.ds(i * 16, 16)] = (hard - sf) + sf
        return carry

    lax.fori_loop(0, _SEG // 16, wr_body, 0)
    pltpu.sync_copy(sel_v, sel_hbm.at[row, pl.ds(off, _SEG)])

    @pl.when(seg == 0)
    def _():
        ek_v[...] = sigsum16
        pltpu.sync_copy(ek_v, ek_hbm.at[row])


@functools.partial(jax.jit, static_argnames=())
def kernel(token_embeddings, W1, b1, W2, b2):
    B, S, D = token_embeddings.shape
    H = W1.shape[1]
    N = B * S
    TS = 4096
    x2 = token_embeddings.reshape(N, D)
    b1r = b1.reshape(1, H)
    b2r = b2.reshape(1, 1)

    logits = pl.pallas_call(
        _score_body,
        grid=(N // TS,),
        in_specs=[
            pl.BlockSpec((TS, D), lambda i: (i, 0)),
            pl.BlockSpec((D, H), lambda i: (0, 0)),
            pl.BlockSpec((1, H), lambda i: (0, 0)),
            pl.BlockSpec((H, 1), lambda i: (0, 0)),
            pl.BlockSpec((1, 1), lambda i: (0, 0)),
        ],
        out_specs=pl.BlockSpec((TS, 1), lambda i: (i, 0)),
        out_shape=jax.ShapeDtypeStruct((N, 1), jnp.float32),
    )(x2, W1, b1r, W2, b2r)

    sc_select = pl.kernel(
        _sc_select_body,
        out_type=[
            jax.ShapeDtypeStruct((B, S), jnp.float32),
            jax.ShapeDtypeStruct((B, 16), jnp.float32),
        ],
        mesh=plsc.VectorSubcoreMesh(core_axis_name="c", subcore_axis_name="s",
                                    num_cores=2, num_subcores=16),
        scratch_types=[
            pltpu.VMEM((_SEG,), jnp.float32),
            pltpu.VMEM((_SEG,), jnp.float32),
            pltpu.VMEM((_SEG,), jnp.int32),
            pltpu.VMEM((_SEG,), jnp.int32),
            pltpu.VMEM((_SEG,), jnp.float32),
            pltpu.VMEM((_ROW,), jnp.float32),
            pltpu.VMEM((8 * _ROW,), jnp.float32),
            pltpu.VMEM((16,), jnp.float32),
            pltpu.VMEM_SHARED((2 * _SLOT,), jnp.float32),
        ],
    )
    sel, ek16 = sc_select(logits.reshape(B, S))

    filtered = pl.pallas_call(
        _apply_body,
        grid=(N // TS,),
        in_specs=[
            pl.BlockSpec((TS, D), lambda i: (i, 0)),
            pl.BlockSpec((TS, 1), lambda i: (i, 0)),
        ],
        out_specs=pl.BlockSpec((TS, D), lambda i: (i, 0)),
        out_shape=jax.ShapeDtypeStruct((N, D), jnp.float32),
    )(x2, sel.reshape(N, 1))

    return (filtered.reshape(B, S, D), sel, ek16[:, 0])


# speedup vs baseline: 1.0895x; 1.0895x over previous
import functools

import jax
import jax.numpy as jnp
from jax import lax
from jax.experimental import pallas as pl
from jax.experimental.pallas import tpu as pltpu
from jax.experimental.pallas import tpu_sc as plsc

_SEG = 1024
_ROW = 240
_SLOT = 16 * _ROW


def _score_body(x_ref, w1_ref, b1_ref, w2_ref, b2_ref, out_ref):
    x = x_ref[...]
    h = jnp.dot(x, w1_ref[...], preferred_element_type=jnp.float32) + b1_ref[...]
    h = jnp.maximum(h, 0.0)
    out_ref[...] = jnp.dot(h, w2_ref[...], preferred_element_type=jnp.float32) + b2_ref[...]


def _apply_body(x_ref, s_ref, o_ref):
    o_ref[...] = x_ref[...] * s_ref[...]


def _sc_select_body(l_hbm, sel_hbm, ek_hbm,
                    l_v, e_v, sb_v, tp_v, sel_v, stbig_v, redbig_v, ek_v, part):
    c = lax.axis_index("c")
    s = lax.axis_index("s")
    rg = s // 8
    seg = s % 8
    row = c * 2 + rg
    base = rg * 8
    off = seg * _SEG
    lane = lax.iota(jnp.int32, 16)
    zero16 = jnp.zeros((16,), jnp.float32)

    def _allmax(v):
        for sh in (1, 2, 4, 8):
            v = jnp.maximum(v, jnp.take(v, lane ^ sh))
        return v

    def _allsum(v):
        for sh in (1, 2, 4, 8):
            v = v + jnp.take(v, lane ^ sh)
        return v

    def _staged(accs, slot, op):
        n = len(accs)
        for j, a in enumerate(accs):
            stbig_v[pl.ds(j * 16, 16)] = a
        pltpu.sync_copy(stbig_v.at[pl.ds(0, n * 16)],
                        part.at[pl.ds(slot * _SLOT + s * _ROW, n * 16)])
        plsc.subcore_barrier()
        pltpu.sync_copy(part.at[pl.ds(slot * _SLOT + base * _ROW, 8 * _ROW)],
                        redbig_v)
        outs = []
        for j in range(n):
            a = redbig_v[pl.ds(j * 16, 16)]
            for g in range(1, 8):
                a = op(a, redbig_v[pl.ds(g * _ROW + j * 16, 16)])
            outs.append(_allsum(a) if op is jnp.add else _allmax(a))
        return outs

    pltpu.sync_copy(l_hbm.at[row, pl.ds(off, _SEG)], l_v)

    def mx_body(i, acc):
        return jnp.maximum(acc, l_v[pl.ds(i * 16, 16)])

    mx = lax.fori_loop(0, _SEG // 16, mx_body,
                       jnp.full((16,), -jnp.inf, jnp.float32))
    (m16,) = _staged([mx], 0, jnp.maximum)

    def ez_body(i, carry):
        zacc, gacc = carry
        x = l_v[pl.ds(i * 16, 16)]
        e = jnp.exp(x - m16)
        e_v[pl.ds(i * 16, 16)] = e
        en = jnp.exp(-jnp.abs(x))
        sig = jnp.where(x >= 0, 1.0 / (1.0 + en), en / (1.0 + en))
        return zacc + e, gacc + sig

    zacc, gacc = lax.fori_loop(0, _SEG // 16, ez_body, (zero16, zero16))
    z16, sigsum16 = _staged([zacc, gacc], 1, jnp.add)
    k16 = sigsum16.astype(jnp.int32).astype(jnp.float32)

    def sbb(i, carry):
        e = e_v[pl.ds(i * 16, 16)]
        sb_v[pl.ds(i * 16, 16)] = lax.bitcast_convert_type(e / z16, jnp.int32)
        return carry

    lax.fori_loop(0, _SEG // 16, sbb, 0)

    P = jnp.zeros((16,), jnp.int32)
    slot = 0
    for sh, nmax in [(28, 7)] + [(24 - 4 * i, 15) for i in range(7)]:
        probes = [P + (n << sh) for n in range(1, nmax + 1)]

        def cbody(i, accs, probes=probes):
            v = sb_v[pl.ds(i * 16, 16)]
            return tuple(a + jnp.where(v >= t, 1.0, 0.0)
                         for a, t in zip(accs, probes))

        accs = lax.fori_loop(0, _SEG // 16, cbody, (zero16,) * nmax)
        cnts = _staged(list(accs), slot, jnp.add)
        slot = 1 - slot
        nsel = jnp.zeros((16,), jnp.int32)
        for cn in cnts:
            nsel = nsel + jnp.where(cn >= k16, 1, 0)
        P = P + (nsel << sh)
    thr = P

    def gt_body(i, acc):
        v = sb_v[pl.ds(i * 16, 16)]
        p = off + i * 16 + lane
        tp_v[pl.ds(i * 16, 16)] = jnp.where(v == thr, p, 8192)
        return acc + jnp.where(v > thr, 1.0, 0.0)

    (n_gt,) = _staged([lax.fori_loop(0, _SEG // 16, gt_body, zero16)],
                      slot, jnp.add)
    slot = 1 - slot

    Q = jnp.zeros((16,), jnp.int32)
    for sh, nmax in [(12, 1), (8, 15), (4, 15), (0, 15)]:
        probes = [Q + (n << sh) for n in range(1, nmax + 1)]

        def qbody(i, accs, probes=probes):
            tp = tp_v[pl.ds(i * 16, 16)]
            return tuple(a + jnp.where(tp < t, 1.0, 0.0)
                         for a, t in zip(accs, probes))

        accs = lax.fori_loop(0, _SEG // 16, qbody, (zero16,) * nmax)
        cnts = _staged(list(accs), slot, jnp.add)
        slot = 1 - slot
        nsel = jnp.zeros((16,), jnp.int32)
        for cn in cnts:
            nsel = nsel + jnp.where(n_gt + cn < k16, 1, 0)
        Q = Q + (nsel << sh)
    limit = Q + 1

    def wr_body(i, carry):
        sb = sb_v[pl.ds(i * 16, 16)]
        sf = lax.bitcast_convert_type(sb, jnp.float32)
        p = off + i * 16 + lane
        hardb = (sb > thr) | ((sb == thr) & (p < limit))
        hard = jnp.where(hardb, 1.0, 0.0)
        sel_v[pl.ds(i * 16, 16)] = (hard - sf) + sf
        return carry

    lax.fori_loop(0, _SEG // 16, wr_body, 0)
    pltpu.sync_copy(sel_v, sel_hbm.at[row, pl.ds(off, _SEG)])

    @pl.when(seg == 0)
    def _():
        ek_v[...] = sigsum16
        pltpu.sync_copy(ek_v, ek_hbm.at[row])


@functools.partial(jax.jit, static_argnames=())
def kernel(token_embeddings, W1, b1, W2, b2):
    B, S, D = token_embeddings.shape
    H = W1.shape[1]
    N = B * S
    TS = 4096
    x2 = token_embeddings.reshape(N, D)
    b1r = b1.reshape(1, H)
    b2r = b2.reshape(1, 1)

    logits = pl.pallas_call(
        _score_body,
        grid=(N // TS,),
        in_specs=[
            pl.BlockSpec((TS, D), lambda i: (i, 0)),
            pl.BlockSpec((D, H), lambda i: (0, 0)),
            pl.BlockSpec((1, H), lambda i: (0, 0)),
            pl.BlockSpec((H, 1), lambda i: (0, 0)),
            pl.BlockSpec((1, 1), lambda i: (0, 0)),
        ],
        out_specs=pl.BlockSpec((TS, 1), lambda i: (i, 0)),
        out_shape=jax.ShapeDtypeStruct((N, 1), jnp.float32),
    )(x2, W1, b1r, W2, b2r)

    sc_select = pl.kernel(
        _sc_select_body,
        out_type=[
            jax.ShapeDtypeStruct((B, S), jnp.float32),
            jax.ShapeDtypeStruct((B, 16), jnp.float32),
        ],
        mesh=plsc.VectorSubcoreMesh(core_axis_name="c", subcore_axis_name="s",
                                    num_cores=2, num_subcores=16),
        scratch_types=[
            pltpu.VMEM((_SEG,), jnp.float32),
            pltpu.VMEM((_SEG,), jnp.float32),
            pltpu.VMEM((_SEG,), jnp.int32),
            pltpu.VMEM((_SEG,), jnp.int32),
            pltpu.VMEM((_SEG,), jnp.float32),
            pltpu.VMEM((_ROW,), jnp.float32),
            pltpu.VMEM((8 * _ROW,), jnp.float32),
            pltpu.VMEM((16,), jnp.float32),
            pltpu.VMEM_SHARED((2 * _SLOT,), jnp.float32),
        ],
    )
    sel, ek16 = sc_select(logits.reshape(B, S))

    filtered = pl.pallas_call(
        _apply_body,
        grid=(N // TS,),
        in_specs=[
            pl.BlockSpec((TS, D), lambda i: (i, 0)),
            pl.BlockSpec((TS, 1), lambda i: (i, 0)),
        ],
        out_specs=pl.BlockSpec((TS, D), lambda i: (i, 0)),
        out_shape=jax.ShapeDtypeStruct((N, D), jnp.float32),
    )(x2, sel.reshape(N, 1))

    return (filtered.reshape(B, S, D), sel, ek16[:, 0])
